# baseline (speedup 1.0000x reference)
import functools

import jax
import jax.numpy as jnp
from jax import lax
from jax.experimental import pallas as pl
from jax.experimental.pallas import tpu as pltpu
from jax.experimental.pallas import tpu_sc as plsc

N = 10000
NC = 2
NS = 16
NW = NC * NS
CHUNK = 64
TOTAL_CHUNKS = 5120
E_PAD = TOTAL_CHUNKS * CHUNK
CA = 208
CB = TOTAL_CHUNKS // NS - CA
ROWS_PER_TILE = 640
N_PAD = NS * ROWS_PER_TILE
SLAB = 64
IB = 16
NCLASS = 40
C_PAD = 128

_mesh = plsc.VectorSubcoreMesh(core_axis_name="c", subcore_axis_name="s")


def _make_sc_aggregate(D):
    @functools.partial(
        pl.kernel,
        out_type=jax.ShapeDtypeStruct((NC, N_PAD, D), jnp.float32),
        mesh=_mesh,
        scratch_types=[
            pltpu.VMEM((IB, CHUNK), jnp.int32),
            pltpu.VMEM((IB, CHUNK), jnp.int32),
            pltpu.VMEM((CHUNK, D), jnp.float32),
            pltpu.VMEM((CHUNK, D), jnp.float32),
            pltpu.VMEM((CHUNK, D), jnp.float32),
            pltpu.VMEM((CHUNK, D), jnp.float32),
            pltpu.VMEM((16, D), jnp.float32),
            pltpu.VMEM_SHARED((N_PAD, D), jnp.float32),
            pltpu.SemaphoreType.DMA,
            pltpu.SemaphoreType.DMA,
            pltpu.SemaphoreType.DMA,
            pltpu.SemaphoreType.DMA,
        ],
    )
    def sc_aggregate(sup_hbm, edges_hbm, out_hbm, idx_s, idx_d, gbuf0, gbuf1,
                     gbuf2, gbuf3, zbuf, acc, sem0, sem1, sem2, sem3):
        c = lax.axis_index("c")
        s = lax.axis_index("s")
        start = jnp.where(c == 0, s * CA, NS * CA + s * CB)
        nblk = jnp.where(c == 0, CA // IB, CB // IB)
        row0 = s * ROWS_PER_TILE
        gbufs = (gbuf0, gbuf1, gbuf2, gbuf3)
        sems = (sem0, sem1, sem2, sem3)

        zero = jnp.zeros((16,), jnp.float32)
        for r in range(16):
            for col in range(D // 16):
                zbuf[r, pl.ds(col * 16, 16)] = zero

        @pl.loop(0, ROWS_PER_TILE, step=16)
        def _zero(k):
            pltpu.sync_copy(zbuf, acc.at[pl.ds(row0 + k, 16)])

        plsc.subcore_barrier()

        @pl.loop(0, nblk)
        def _blk(b):
            base = (start + b * IB) * 1
            pltpu.sync_copy(edges_hbm.at[0, pl.ds(base, IB)], idx_s)
            pltpu.sync_copy(edges_hbm.at[1, pl.ds(base, IB)], idx_d)
            descs = [None, None, None, None]
            for f in range(3):
                descs[f] = pltpu.async_copy(sup_hbm.at[idx_s.at[f]], gbufs[f],
                                            sems[f])
            for jj in range(IB):
                p = jj % 4
                if jj + 3 < IB:
                    q = (jj + 3) % 4
                    descs[q] = pltpu.async_copy(sup_hbm.at[idx_s.at[jj + 3]],
                                                gbufs[q], sems[q])
                descs[p].wait()
                pltpu.sync_copy(gbufs[p], acc.at[idx_d.at[jj]], add=True)

        plsc.subcore_barrier()

        pltpu.sync_copy(acc.at[pl.ds(row0, ROWS_PER_TILE)],
                        out_hbm.at[c, pl.ds(row0, ROWS_PER_TILE)])

    return sc_aggregate


_sc_aggregate_128 = _make_sc_aggregate(128)

_BM = 2000


def _mm_body(x_ref, w_ref, o_ref):
    o_ref[...] = jnp.dot(x_ref[...], w_ref[...],
                         preferred_element_type=jnp.float32)


def _mm(x, w):
    m, k = x.shape
    n = w.shape[1]
    return pl.pallas_call(
        _mm_body,
        grid=(m // _BM,),
        in_specs=[pl.BlockSpec((_BM, k), lambda i: (i, 0)),
                  pl.BlockSpec((k, n), lambda i: (0, 0))],
        out_specs=pl.BlockSpec((_BM, n), lambda i: (i, 0)),
        out_shape=jax.ShapeDtypeStruct((m, n), jnp.float32),
    )(x, w)


def _combine_body(x_ref, a0_ref, a1_ref, ws_ref, b_ref, wn_ref, xn_ref,
                  sn_ref):
    xn = (a0_ref[...] + a1_ref[...] + b_ref[...]
          + jnp.dot(x_ref[...], ws_ref[...],
                    preferred_element_type=jnp.float32))
    xn_ref[...] = xn
    sn_ref[...] = jnp.dot(xn, wn_ref[...], preferred_element_type=jnp.float32)


def _combine(x, a0, a1, ws, b, wn):
    m, k = x.shape
    d = ws.shape[1]
    dn = wn.shape[1]
    return pl.pallas_call(
        _combine_body,
        grid=(m // _BM,),
        in_specs=[pl.BlockSpec((_BM, k), lambda i: (i, 0)),
                  pl.BlockSpec((_BM, d), lambda i: (i, 0)),
                  pl.BlockSpec((_BM, d), lambda i: (i, 0)),
                  pl.BlockSpec((k, d), lambda i: (0, 0)),
                  pl.BlockSpec((1, d), lambda i: (0, 0)),
                  pl.BlockSpec((d, dn), lambda i: (0, 0))],
        out_specs=[pl.BlockSpec((_BM, d), lambda i: (i, 0)),
                   pl.BlockSpec((_BM, dn), lambda i: (i, 0))],
        out_shape=[jax.ShapeDtypeStruct((m, d), jnp.float32),
                   jax.ShapeDtypeStruct((m, dn), jnp.float32)],
    )(x, a0, a1, ws, b.reshape(1, -1), wn)


def _final_body(x_ref, a0_ref, a1_ref, ws_ref, b_ref, o_ref):
    z = (a0_ref[...] + a1_ref[...] + b_ref[...]
         + jnp.dot(x_ref[...], ws_ref[...],
                   preferred_element_type=jnp.float32))
    col = lax.broadcasted_iota(jnp.int32, z.shape, 1)
    z = jnp.where(col < NCLASS, z, -1e30)
    m = jnp.max(z, axis=1, keepdims=True)
    e = jnp.exp(z - m)
    o_ref[...] = z - m - jnp.log(jnp.sum(e, axis=1, keepdims=True))


def _final(x, a0, a1, ws, b):
    m, k = x.shape
    d = ws.shape[1]
    return pl.pallas_call(
        _final_body,
        grid=(m // _BM,),
        in_specs=[pl.BlockSpec((_BM, k), lambda i: (i, 0)),
                  pl.BlockSpec((_BM, d), lambda i: (i, 0)),
                  pl.BlockSpec((_BM, d), lambda i: (i, 0)),
                  pl.BlockSpec((k, d), lambda i: (0, 0)),
                  pl.BlockSpec((1, d), lambda i: (0, 0))],
        out_specs=pl.BlockSpec((_BM, d), lambda i: (i, 0)),
        out_shape=jax.ShapeDtypeStruct((m, d), jnp.float32),
    )(x, a0, a1, ws, b.reshape(1, -1))


def kernel(fea, edge_index, W_in, Ws_in, b_in, W_mid, Ws_mid, b_mid, W_out,
           Ws_out, b_out):
    e = edge_index.shape[1]
    pad = E_PAD - e
    src = jnp.concatenate([edge_index[0], jnp.zeros((pad,), jnp.int32)])
    dst = jnp.concatenate([edge_index[1], jnp.full((pad,), N, jnp.int32)])
    edges = jnp.stack([src, dst]).reshape(2, TOTAL_CHUNKS, CHUNK)

    wo_p = jnp.pad(W_out, ((0, 0), (0, C_PAD - NCLASS)))
    wso_p = jnp.pad(Ws_out, ((0, 0), (0, C_PAD - NCLASS)))
    bo_p = jnp.pad(b_out, (0, C_PAD - NCLASS))

    s1 = _mm(fea, W_in)
    agg1 = _sc_aggregate_128(s1, edges)
    x1, s2 = _combine(fea, agg1[0], agg1[1], Ws_in, b_in, W_mid)
    agg2 = _sc_aggregate_128(s2, edges)
    x2, s3 = _combine(x1, agg2[0], agg2[1], Ws_mid, b_mid, wo_p)
    agg3 = _sc_aggregate_128(s3, edges)
    out = _final(x2, agg3[0], agg3[1], wso_p, bo_p)
    return out[:, :NCLASS]


# speedup vs baseline: 1.0745x; 1.0745x over previous
import functools

import jax
import jax.numpy as jnp
from jax import lax
from jax.experimental import pallas as pl
from jax.experimental.pallas import tpu as pltpu
from jax.experimental.pallas import tpu_sc as plsc

N = 10000
NC = 2
NS = 16
NW = NC * NS
CHUNK = 64
TOTAL_CHUNKS = 5120
E_PAD = TOTAL_CHUNKS * CHUNK
CA = 272
CB = TOTAL_CHUNKS // NS - CA
ROWS_PER_TILE = 640
N_PAD = NS * ROWS_PER_TILE
SLAB = 64
IB = 16
NCLASS = 40
C_PAD = 128

_mesh = plsc.VectorSubcoreMesh(core_axis_name="c", subcore_axis_name="s")


def _make_sc_aggregate(D):
    @functools.partial(
        pl.kernel,
        out_type=jax.ShapeDtypeStruct((NC, N_PAD, D), jnp.float32),
        mesh=_mesh,
        scratch_types=[
            pltpu.VMEM((IB, CHUNK), jnp.int32),
            pltpu.VMEM((IB, CHUNK), jnp.int32),
            pltpu.VMEM((CHUNK, D), jnp.float32),
            pltpu.VMEM((CHUNK, D), jnp.float32),
            pltpu.VMEM((CHUNK, D), jnp.float32),
            pltpu.VMEM((CHUNK, D), jnp.float32),
            pltpu.VMEM((16, D), jnp.float32),
            pltpu.VMEM_SHARED((N_PAD, D), jnp.float32),
            pltpu.SemaphoreType.DMA,
            pltpu.SemaphoreType.DMA,
            pltpu.SemaphoreType.DMA,
            pltpu.SemaphoreType.DMA,
        ],
    )
    def sc_aggregate(sup_hbm, edges_hbm, out_hbm, idx_s, idx_d, gbuf0, gbuf1,
                     gbuf2, gbuf3, zbuf, acc, sem0, sem1, sem2, sem3):
        c = lax.axis_index("c")
        s = lax.axis_index("s")
        start = jnp.where(c == 0, s * CA, NS * CA + s * CB)
        nblk = jnp.where(c == 0, CA // IB, CB // IB)
        row0 = s * ROWS_PER_TILE
        gbufs = (gbuf0, gbuf1, gbuf2, gbuf3)
        sems = (sem0, sem1, sem2, sem3)

        zero = jnp.zeros((16,), jnp.float32)
        for r in range(16):
            for col in range(D // 16):
                zbuf[r, pl.ds(col * 16, 16)] = zero

        @pl.loop(0, ROWS_PER_TILE, step=16)
        def _zero(k):
            pltpu.sync_copy(zbuf, acc.at[pl.ds(row0 + k, 16)])

        plsc.subcore_barrier()

        @pl.loop(0, nblk)
        def _blk(b):
            base = (start + b * IB) * 1
            pltpu.sync_copy(edges_hbm.at[0, pl.ds(base, IB)], idx_s)
            pltpu.sync_copy(edges_hbm.at[1, pl.ds(base, IB)], idx_d)
            descs = [None, None, None, None]
            for f in range(3):
                descs[f] = pltpu.async_copy(sup_hbm.at[idx_s.at[f]], gbufs[f],
                                            sems[f])
            for jj in range(IB):
                p = jj % 4
                if jj + 3 < IB:
                    q = (jj + 3) % 4
                    descs[q] = pltpu.async_copy(sup_hbm.at[idx_s.at[jj + 3]],
                                                gbufs[q], sems[q])
                descs[p].wait()
                pltpu.sync_copy(gbufs[p], acc.at[idx_d.at[jj]], add=True)

        plsc.subcore_barrier()

        pltpu.sync_copy(acc.at[pl.ds(row0, ROWS_PER_TILE)],
                        out_hbm.at[c, pl.ds(row0, ROWS_PER_TILE)])

    return sc_aggregate


_sc_aggregate_128 = _make_sc_aggregate(128)

_BM = 2000


def _mm_body(x_ref, w_ref, o_ref):
    o_ref[...] = jnp.dot(x_ref[...], w_ref[...],
                         preferred_element_type=jnp.float32)


def _mm(x, w):
    m, k = x.shape
    n = w.shape[1]
    return pl.pallas_call(
        _mm_body,
        grid=(m // _BM,),
        in_specs=[pl.BlockSpec((_BM, k), lambda i: (i, 0)),
                  pl.BlockSpec((k, n), lambda i: (0, 0))],
        out_specs=pl.BlockSpec((_BM, n), lambda i: (i, 0)),
        out_shape=jax.ShapeDtypeStruct((m, n), jnp.float32),
    )(x, w)


def _combine_body(x_ref, a0_ref, a1_ref, ws_ref, b_ref, wn_ref, xn_ref,
                  sn_ref):
    xn = (a0_ref[...] + a1_ref[...] + b_ref[...]
          + jnp.dot(x_ref[...], ws_ref[...],
                    preferred_element_type=jnp.float32))
    xn_ref[...] = xn
    sn_ref[...] = jnp.dot(xn, wn_ref[...], preferred_element_type=jnp.float32)


def _combine(x, a0, a1, ws, b, wn):
    m, k = x.shape
    d = ws.shape[1]
    dn = wn.shape[1]
    return pl.pallas_call(
        _combine_body,
        grid=(m // _BM,),
        in_specs=[pl.BlockSpec((_BM, k), lambda i: (i, 0)),
                  pl.BlockSpec((_BM, d), lambda i: (i, 0)),
                  pl.BlockSpec((_BM, d), lambda i: (i, 0)),
                  pl.BlockSpec((k, d), lambda i: (0, 0)),
                  pl.BlockSpec((1, d), lambda i: (0, 0)),
                  pl.BlockSpec((d, dn), lambda i: (0, 0))],
        out_specs=[pl.BlockSpec((_BM, d), lambda i: (i, 0)),
                   pl.BlockSpec((_BM, dn), lambda i: (i, 0))],
        out_shape=[jax.ShapeDtypeStruct((m, d), jnp.float32),
                   jax.ShapeDtypeStruct((m, dn), jnp.float32)],
    )(x, a0, a1, ws, b.reshape(1, -1), wn)


def _final_body(x_ref, a0_ref, a1_ref, ws_ref, b_ref, o_ref):
    z = (a0_ref[...] + a1_ref[...] + b_ref[...]
         + jnp.dot(x_ref[...], ws_ref[...],
                   preferred_element_type=jnp.float32))
    col = lax.broadcasted_iota(jnp.int32, z.shape, 1)
    z = jnp.where(col < NCLASS, z, -1e30)
    m = jnp.max(z, axis=1, keepdims=True)
    e = jnp.exp(z - m)
    o_ref[...] = z - m - jnp.log(jnp.sum(e, axis=1, keepdims=True))


def _final(x, a0, a1, ws, b):
    m, k = x.shape
    d = ws.shape[1]
    return pl.pallas_call(
        _final_body,
        grid=(m // _BM,),
        in_specs=[pl.BlockSpec((_BM, k), lambda i: (i, 0)),
                  pl.BlockSpec((_BM, d), lambda i: (i, 0)),
                  pl.BlockSpec((_BM, d), lambda i: (i, 0)),
                  pl.BlockSpec((k, d), lambda i: (0, 0)),
                  pl.BlockSpec((1, d), lambda i: (0, 0))],
        out_specs=pl.BlockSpec((_BM, d), lambda i: (i, 0)),
        out_shape=jax.ShapeDtypeStruct((m, d), jnp.float32),
    )(x, a0, a1, ws, b.reshape(1, -1))


def kernel(fea, edge_index, W_in, Ws_in, b_in, W_mid, Ws_mid, b_mid, W_out,
           Ws_out, b_out):
    e = edge_index.shape[1]
    pad = E_PAD - e
    src = jnp.concatenate([edge_index[0], jnp.zeros((pad,), jnp.int32)])
    dst = jnp.concatenate([edge_index[1], jnp.full((pad,), N, jnp.int32)])
    edges = jnp.stack([src, dst]).reshape(2, TOTAL_CHUNKS, CHUNK)

    wo_p = jnp.pad(W_out, ((0, 0), (0, C_PAD - NCLASS)))
    wso_p = jnp.pad(Ws_out, ((0, 0), (0, C_PAD - NCLASS)))
    bo_p = jnp.pad(b_out, (0, C_PAD - NCLASS))

    s1 = _mm(fea, W_in)
    agg1 = _sc_aggregate_128(s1, edges)
    x1, s2 = _combine(fea, agg1[0], agg1[1], Ws_in, b_in, W_mid)
    agg2 = _sc_aggregate_128(s2, edges)
    x2, s3 = _combine(x1, agg2[0], agg2[1], Ws_mid, b_mid, wo_p)
    agg3 = _sc_aggregate_128(s3, edges)
    out = _final(x2, agg3[0], agg3[1], wso_p, bo_p)
    return out[:, :NCLASS]


# speedup vs baseline: 1.1815x; 1.0996x over previous
import functools

import jax
import jax.numpy as jnp
from jax import lax
from jax.experimental import pallas as pl
from jax.experimental.pallas import tpu as pltpu
from jax.experimental.pallas import tpu_sc as plsc

N = 10000
NC = 2
NS = 16
NW = NC * NS
CHUNK = 64
TOTAL_CHUNKS = 5120
E_PAD = TOTAL_CHUNKS * CHUNK
CA = 304
CB = TOTAL_CHUNKS // NS - CA
ROWS_PER_TILE = 640
N_PAD = NS * ROWS_PER_TILE
SLAB = 64
IB = 16
NCLASS = 40
C_PAD = 128

_mesh = plsc.VectorSubcoreMesh(core_axis_name="c", subcore_axis_name="s")


def _make_sc_aggregate(D):
    @functools.partial(
        pl.kernel,
        out_type=jax.ShapeDtypeStruct((NC, N_PAD, D), jnp.float32),
        mesh=_mesh,
        scratch_types=[
            pltpu.VMEM((IB, CHUNK), jnp.int32),
            pltpu.VMEM((IB, CHUNK), jnp.int32),
            pltpu.VMEM((CHUNK, D), jnp.float32),
            pltpu.VMEM((CHUNK, D), jnp.float32),
            pltpu.VMEM((CHUNK, D), jnp.float32),
            pltpu.VMEM((CHUNK, D), jnp.float32),
            pltpu.VMEM((16, D), jnp.float32),
            pltpu.VMEM_SHARED((N_PAD, D), jnp.float32),
            pltpu.SemaphoreType.DMA,
            pltpu.SemaphoreType.DMA,
            pltpu.SemaphoreType.DMA,
            pltpu.SemaphoreType.DMA,
        ],
    )
    def sc_aggregate(sup_hbm, edges_hbm, out_hbm, idx_s, idx_d, gbuf0, gbuf1,
                     gbuf2, gbuf3, zbuf, acc, sem0, sem1, sem2, sem3):
        c = lax.axis_index("c")
        s = lax.axis_index("s")
        start = jnp.where(c == 0, s * CA, NS * CA + s * CB)
        nblk = jnp.where(c == 0, CA // IB, CB // IB)
        row0 = s * ROWS_PER_TILE
        gbufs = (gbuf0, gbuf1, gbuf2, gbuf3)
        sems = (sem0, sem1, sem2, sem3)

        zero = jnp.zeros((16,), jnp.float32)
        for r in range(16):
            for col in range(D // 16):
                zbuf[r, pl.ds(col * 16, 16)] = zero

        @pl.loop(0, ROWS_PER_TILE, step=16)
        def _zero(k):
            pltpu.sync_copy(zbuf, acc.at[pl.ds(row0 + k, 16)])

        plsc.subcore_barrier()

        @pl.loop(0, nblk)
        def _blk(b):
            base = (start + b * IB) * 1
            pltpu.sync_copy(edges_hbm.at[0, pl.ds(base, IB)], idx_s)
            pltpu.sync_copy(edges_hbm.at[1, pl.ds(base, IB)], idx_d)
            descs = [None, None, None, None]
            for f in range(3):
                descs[f] = pltpu.async_copy(sup_hbm.at[idx_s.at[f]], gbufs[f],
                                            sems[f])
            for jj in range(IB):
                p = jj % 4
                if jj + 3 < IB:
                    q = (jj + 3) % 4
                    descs[q] = pltpu.async_copy(sup_hbm.at[idx_s.at[jj + 3]],
                                                gbufs[q], sems[q])
                descs[p].wait()
                pltpu.sync_copy(gbufs[p], acc.at[idx_d.at[jj]], add=True)

        plsc.subcore_barrier()

        pltpu.sync_copy(acc.at[pl.ds(row0, ROWS_PER_TILE)],
                        out_hbm.at[c, pl.ds(row0, ROWS_PER_TILE)])

    return sc_aggregate


_sc_aggregate_128 = _make_sc_aggregate(128)

_BM = 2000


def _mm_body(x_ref, w_ref, o_ref):
    o_ref[...] = jnp.dot(x_ref[...], w_ref[...],
                         preferred_element_type=jnp.float32)


def _mm(x, w):
    m, k = x.shape
    n = w.shape[1]
    return pl.pallas_call(
        _mm_body,
        grid=(m // _BM,),
        in_specs=[pl.BlockSpec((_BM, k), lambda i: (i, 0)),
                  pl.BlockSpec((k, n), lambda i: (0, 0))],
        out_specs=pl.BlockSpec((_BM, n), lambda i: (i, 0)),
        out_shape=jax.ShapeDtypeStruct((m, n), jnp.float32),
    )(x, w)


def _combine_body(x_ref, a0_ref, a1_ref, ws_ref, b_ref, wn_ref, xn_ref,
                  sn_ref):
    xn = (a0_ref[...] + a1_ref[...] + b_ref[...]
          + jnp.dot(x_ref[...], ws_ref[...],
                    preferred_element_type=jnp.float32))
    xn_ref[...] = xn
    sn_ref[...] = jnp.dot(xn, wn_ref[...], preferred_element_type=jnp.float32)


def _combine(x, a0, a1, ws, b, wn):
    m, k = x.shape
    d = ws.shape[1]
    dn = wn.shape[1]
    return pl.pallas_call(
        _combine_body,
        grid=(m // _BM,),
        in_specs=[pl.BlockSpec((_BM, k), lambda i: (i, 0)),
                  pl.BlockSpec((_BM, d), lambda i: (i, 0)),
                  pl.BlockSpec((_BM, d), lambda i: (i, 0)),
                  pl.BlockSpec((k, d), lambda i: (0, 0)),
                  pl.BlockSpec((1, d), lambda i: (0, 0)),
                  pl.BlockSpec((d, dn), lambda i: (0, 0))],
        out_specs=[pl.BlockSpec((_BM, d), lambda i: (i, 0)),
                   pl.BlockSpec((_BM, dn), lambda i: (i, 0))],
        out_shape=[jax.ShapeDtypeStruct((m, d), jnp.float32),
                   jax.ShapeDtypeStruct((m, dn), jnp.float32)],
    )(x, a0, a1, ws, b.reshape(1, -1), wn)


def _final_body(x_ref, a0_ref, a1_ref, ws_ref, b_ref, o_ref):
    z = (a0_ref[...] + a1_ref[...] + b_ref[...]
         + jnp.dot(x_ref[...], ws_ref[...],
                   preferred_element_type=jnp.float32))
    col = lax.broadcasted_iota(jnp.int32, z.shape, 1)
    z = jnp.where(col < NCLASS, z, -1e30)
    m = jnp.max(z, axis=1, keepdims=True)
    e = jnp.exp(z - m)
    o_ref[...] = z - m - jnp.log(jnp.sum(e, axis=1, keepdims=True))


def _final(x, a0, a1, ws, b):
    m, k = x.shape
    d = ws.shape[1]
    return pl.pallas_call(
        _final_body,
        grid=(m // _BM,),
        in_specs=[pl.BlockSpec((_BM, k), lambda i: (i, 0)),
                  pl.BlockSpec((_BM, d), lambda i: (i, 0)),
                  pl.BlockSpec((_BM, d), lambda i: (i, 0)),
                  pl.BlockSpec((k, d), lambda i: (0, 0)),
                  pl.BlockSpec((1, d), lambda i: (0, 0))],
        out_specs=pl.BlockSpec((_BM, d), lambda i: (i, 0)),
        out_shape=jax.ShapeDtypeStruct((m, d), jnp.float32),
    )(x, a0, a1, ws, b.reshape(1, -1))


def kernel(fea, edge_index, W_in, Ws_in, b_in, W_mid, Ws_mid, b_mid, W_out,
           Ws_out, b_out):
    e = edge_index.shape[1]
    pad = E_PAD - e
    src = jnp.concatenate([edge_index[0], jnp.zeros((pad,), jnp.int32)])
    dst = jnp.concatenate([edge_index[1], jnp.full((pad,), N, jnp.int32)])
    edges = jnp.stack([src, dst]).reshape(2, TOTAL_CHUNKS, CHUNK)

    wo_p = jnp.pad(W_out, ((0, 0), (0, C_PAD - NCLASS)))
    wso_p = jnp.pad(Ws_out, ((0, 0), (0, C_PAD - NCLASS)))
    bo_p = jnp.pad(b_out, (0, C_PAD - NCLASS))

    s1 = _mm(fea, W_in)
    agg1 = _sc_aggregate_128(s1, edges)
    x1, s2 = _combine(fea, agg1[0], agg1[1], Ws_in, b_in, W_mid)
    agg2 = _sc_aggregate_128(s2, edges)
    x2, s3 = _combine(x1, agg2[0], agg2[1], Ws_mid, b_mid, wo_p)
    agg3 = _sc_aggregate_128(s3, edges)
    out = _final(x2, agg3[0], agg3[1], wso_p, bo_p)
    return out[:, :NCLASS]
